# Initial kernel scaffold; baseline (speedup 1.0000x reference)
import jax
import jax.numpy as jnp
from jax.experimental import pallas as pl


def kernel(x, core_types, target_types, W, a_pair, lin_w, lin_b):
    raise NotImplementedError("write your pallas kernel here")


# speedup vs baseline: 1.8435x; 1.8435x over previous
import functools
import math

import jax
import jax.numpy as jnp
from jax.experimental import pallas as pl

NEG_SLOPE = 0.2


def _body(x_ref, c_ref, t_ref, w_ref, ap_ref, out_ref, att_ref,
          *, nb, k, f, nperm, nclass):
    m = nb * k
    xb = x_ref[0]
    c = c_ref[0]
    t = t_ref[0]
    s0 = jnp.minimum(c, t)
    s1 = jnp.maximum(c, t)
    idx = s0 * nclass - (s0 * (s0 - 1)) // 2 + (s1 - s0)
    lanes = jax.lax.broadcasted_iota(jnp.int32, (m, nperm), 1)
    oh = (idx == lanes).astype(jnp.float32)
    a = jnp.dot(oh, ap_ref[...], preferred_element_type=jnp.float32)
    wh = jax.lax.dot_general(xb, w_ref[...], (((0,), (0,)), ((), ())),
                             preferred_element_type=jnp.float32)
    wa = wh * a
    e = jnp.where(wa >= 0, wa, NEG_SLOPE * wa)
    e3 = e.reshape(nb, k, f)
    mx = jnp.max(e3, axis=1, keepdims=True)
    p = jnp.exp(e3 - mx)
    s = jnp.sum(p, axis=1, keepdims=True)
    att = p / s
    att_ref[0] = att
    h = att * wh.reshape(nb, k, f)
    o = jnp.where(h > 0, h, jnp.exp(jnp.minimum(h, 0.0)) - 1.0)
    out_ref[0] = o.reshape(m, f).T


def kernel(x, core_types, target_types, W, a_pair, lin_w, lin_b):
    b, f, n, k = x.shape
    nperm = a_pair.shape[0]
    nclass = int((math.isqrt(8 * nperm + 1) - 1) // 2)

    nb = 256
    while n % nb:
        nb //= 2
    m = nb * k

    x2 = x.reshape(b, f, n * k)
    c2 = jnp.broadcast_to(core_types[:, :, None], (b, n, k)).reshape(b, n * k, 1)
    t2 = target_types.reshape(b, n * k, 1)

    body = functools.partial(_body, nb=nb, k=k, f=f, nperm=nperm, nclass=nclass)
    out2, att = pl.pallas_call(
        body,
        grid=(b, n // nb),
        in_specs=[
            pl.BlockSpec((1, f, m), lambda i, j: (i, 0, j)),
            pl.BlockSpec((1, m, 1), lambda i, j: (i, j, 0)),
            pl.BlockSpec((1, m, 1), lambda i, j: (i, j, 0)),
            pl.BlockSpec((f, f), lambda i, j: (0, 0)),
            pl.BlockSpec((nperm, f), lambda i, j: (0, 0)),
        ],
        out_specs=[
            pl.BlockSpec((1, f, m), lambda i, j: (i, 0, j)),
            pl.BlockSpec((1, nb, k, f), lambda i, j: (i, j, 0, 0)),
        ],
        out_shape=[
            jax.ShapeDtypeStruct((b, f, n * k), jnp.float32),
            jax.ShapeDtypeStruct((b, n, k, f), jnp.float32),
        ],
    )(x2, c2, t2, W, a_pair)

    return (out2.reshape(b, f, n, k), att)


# speedup vs baseline: 2.5160x; 1.3648x over previous
import functools
import math

import jax
import jax.numpy as jnp
from jax.experimental import pallas as pl

NEG_SLOPE = 0.2


def _body(x_ref, c_ref, t_ref, w_ref, ap_ref, out_ref, att_ref,
          *, nb, k, f, nperm, nclass):
    m = nb * k
    nsq = nclass * nclass
    q_i = jax.lax.broadcasted_iota(jnp.int32, (nsq, nperm), 0)
    p_i = jax.lax.broadcasted_iota(jnp.int32, (nsq, nperm), 1)
    ci = q_i // nclass
    tj = q_i % nclass
    s0 = jnp.minimum(ci, tj)
    s1 = jnp.maximum(ci, tj)
    pidx = s0 * nclass - (s0 * (s0 - 1)) // 2 + (s1 - s0)
    ohq = (pidx == p_i).astype(jnp.float32)
    t64 = jnp.dot(ohq, ap_ref[...], preferred_element_type=jnp.float32)

    idx = c_ref[0] * nclass + t_ref[0]
    q3 = jax.lax.broadcasted_iota(jnp.int32, (nsq, m // 128, 128), 0)
    oht = (idx[None, :, :] == q3).astype(jnp.float32).reshape(nsq, m)

    xb = x_ref[0]
    a = jax.lax.dot_general(oht, t64, (((0,), (0,)), ((), ())),
                            preferred_element_type=jnp.float32)
    wh = jax.lax.dot_general(xb, w_ref[...], (((0,), (0,)), ((), ())),
                             preferred_element_type=jnp.float32)
    wa = wh * a
    e = jnp.where(wa >= 0, wa, NEG_SLOPE * wa)
    e3 = e.reshape(nb, k, f)
    mx = jnp.max(e3, axis=1, keepdims=True)
    p = jnp.exp(e3 - mx)
    s = jnp.sum(p, axis=1, keepdims=True)
    att = p / s
    att_ref[0] = att
    h = att * wh.reshape(nb, k, f)
    o = jnp.where(h > 0, h, jnp.exp(jnp.minimum(h, 0.0)) - 1.0)
    out_ref[0] = o.reshape(m, f).T


def kernel(x, core_types, target_types, W, a_pair, lin_w, lin_b):
    b, f, n, k = x.shape
    nperm = a_pair.shape[0]
    nclass = int((math.isqrt(8 * nperm + 1) - 1) // 2)

    nb = 256
    while n % nb:
        nb //= 2
    m = nb * k

    x2 = x.reshape(b, f, n * k)
    c2 = jnp.broadcast_to(core_types[:, :, None], (b, n, k)).reshape(b, n * k // 128, 128)
    t2 = target_types.reshape(b, n * k // 128, 128)

    body = functools.partial(_body, nb=nb, k=k, f=f, nperm=nperm, nclass=nclass)
    out2, att = pl.pallas_call(
        body,
        grid=(b, n // nb),
        in_specs=[
            pl.BlockSpec((1, f, m), lambda i, j: (i, 0, j)),
            pl.BlockSpec((1, m // 128, 128), lambda i, j: (i, j, 0)),
            pl.BlockSpec((1, m // 128, 128), lambda i, j: (i, j, 0)),
            pl.BlockSpec((f, f), lambda i, j: (0, 0)),
            pl.BlockSpec((nperm, f), lambda i, j: (0, 0)),
        ],
        out_specs=[
            pl.BlockSpec((1, f, m), lambda i, j: (i, 0, j)),
            pl.BlockSpec((1, nb, k, f), lambda i, j: (i, j, 0, 0)),
        ],
        out_shape=[
            jax.ShapeDtypeStruct((b, f, n * k), jnp.float32),
            jax.ShapeDtypeStruct((b, n, k, f), jnp.float32),
        ],
    )(x2, c2, t2, W, a_pair)

    return (out2.reshape(b, f, n, k), att)
